# Initial kernel scaffold; baseline (speedup 1.0000x reference)
import jax
import jax.numpy as jnp
from jax.experimental import pallas as pl


def kernel(x, edge_index, edge_attr, lin1_W, lin1_b, edgelin_W, edgelin_b, eps0, conv0_W, conv0_b, eps1, conv1_W, conv1_b, eps2, conv2_W, conv2_b, pred_W1, pred_b1, pred_W2, pred_b2):
    raise NotImplementedError("write your pallas kernel here")


# speedup vs baseline: 1.9004x; 1.9004x over previous
import functools

import jax
import jax.numpy as jnp
from jax import lax
from jax.experimental import pallas as pl
from jax.experimental.pallas import tpu as pltpu
from jax.experimental.pallas import tpu_sc as plsc

N, E, F, H, ED, O = 10000, 320000, 128, 256, 16, 3
NQ = 4
QW = H // NQ
NTEC = 16
EDGE_BATCH = 128
NB = 157
E_CHUNK = NB * EDGE_BATCH
E_PAD = E_CHUNK * NTEC
ACC_SLICE = 640
N_OUT = ACC_SLICE * NTEC
ROW_BLK = 400
N_BLKS = N // ROW_BLK


def _silu(v):
    return v * jax.nn.sigmoid(v)


@functools.cache
def _make_sc_round(with_edge: bool):
    mesh = plsc.VectorSubcoreMesh(core_axis_name="c", subcore_axis_name="s",
                                  num_cores=2, num_subcores=NTEC)

    out_type = [jax.ShapeDtypeStruct((N_OUT, QW), jnp.float32)
                for _ in range(NQ)]
    scratch = [
        pltpu.VMEM((NB, EDGE_BATCH), jnp.int32),
        pltpu.VMEM((NB, EDGE_BATCH), jnp.int32),
        pltpu.VMEM((EDGE_BATCH, QW), jnp.float32),
        pltpu.VMEM((EDGE_BATCH, QW), jnp.float32),
        pltpu.VMEM_SHARED((N_OUT, QW), jnp.float32),
        pltpu.SemaphoreType.DMA,
    ]

    def body(h0, h1, h2, h3, src3, dst3, zeros, e0, e1, e2, e3,
             a0, a1, a2, a3, sidx, didx, rows, eav, acc, sem):
        cid = lax.axis_index("c")
        sid = lax.axis_index("s")
        ebase = sid * E_CHUNK
        rbase = sid * ACC_SLICE

        pltpu.sync_copy(src3.at[sid], sidx)
        pltpu.sync_copy(dst3.at[sid], didx)

        def run_quarter(h_ref, ea_ref, agg_ref):
            pltpu.sync_copy(zeros.at[pl.ds(rbase, ACC_SLICE)],
                            acc.at[pl.ds(rbase, ACC_SLICE)])
            plsc.subcore_barrier()

            def step(j, carry):
                pltpu.async_copy(h_ref.at[sidx.at[j]], rows, sem).wait()
                if with_edge:
                    pltpu.sync_copy(
                        ea_ref.at[pl.ds(ebase + j * EDGE_BATCH, EDGE_BATCH)],
                        eav)

                    def relu_row(r, c2):
                        for c in range(QW // 16):
                            sl = pl.ds(c * 16, 16)
                            rows[r, sl] = jnp.maximum(
                                rows[r, sl] + eav[r, sl], 0.0)
                        return c2
                    lax.fori_loop(0, EDGE_BATCH, relu_row, 0)
                pltpu.sync_copy(rows, acc.at[didx.at[j]], add=True)
                return carry
            lax.fori_loop(0, NB, step, 0)
            plsc.subcore_barrier()
            pltpu.sync_copy(acc.at[pl.ds(rbase, ACC_SLICE)],
                            agg_ref.at[pl.ds(rbase, ACC_SLICE)])
            plsc.subcore_barrier()

        @pl.when(cid == 0)
        def _():
            run_quarter(h0, e0, a0)
            run_quarter(h1, e1, a1)

        @pl.when(cid == 1)
        def _():
            run_quarter(h2, e2, a2)
            run_quarter(h3, e3, a3)

    return pl.kernel(
        body, out_type=out_type, mesh=mesh, scratch_types=scratch,
        compiler_params=pltpu.CompilerParams(use_tc_tiling_on_sc=False))


def _lin1_body(x, W, b, *hq):
    h = _silu(_silu(jnp.dot(x[...], W[...],
                            preferred_element_type=jnp.float32) + b[...]))
    for q in range(NQ):
        hq[q][...] = h[:, q * QW:(q + 1) * QW]


def _lin1(x, W, b):
    return pl.pallas_call(
        _lin1_body,
        grid=(N_BLKS,),
        in_specs=[
            pl.BlockSpec((ROW_BLK, F), lambda i: (i, 0)),
            pl.BlockSpec((F, H), lambda i: (0, 0)),
            pl.BlockSpec((1, H), lambda i: (0, 0)),
        ],
        out_specs=[pl.BlockSpec((ROW_BLK, QW), lambda i: (i, 0))] * NQ,
        out_shape=[jax.ShapeDtypeStruct((N, QW), jnp.float32)] * NQ,
    )(x, W, b)


_EA_BLK = 512
_EA_BLKS = E_PAD // _EA_BLK


def _ea_body(xe, W, b, *eq):
    v = jnp.dot(xe[...], W[...], preferred_element_type=jnp.float32) + b[...]
    for q in range(NQ):
        eq[q][...] = v[:, q * QW:(q + 1) * QW]


def _edge_lin(edge_attr_pad, W, b):
    return pl.pallas_call(
        _ea_body,
        grid=(_EA_BLKS,),
        in_specs=[
            pl.BlockSpec((_EA_BLK, ED), lambda i: (i, 0)),
            pl.BlockSpec((ED, H), lambda i: (0, 0)),
            pl.BlockSpec((1, H), lambda i: (0, 0)),
        ],
        out_specs=[pl.BlockSpec((_EA_BLK, QW), lambda i: (i, 0))] * NQ,
        out_shape=[jax.ShapeDtypeStruct((E_PAD, QW), jnp.float32)] * NQ,
    )(edge_attr_pad, W, b)


def _conv_body(*refs):
    hq = refs[0:NQ]
    aq = refs[NQ:2 * NQ]
    epsr, W, b = refs[2 * NQ:2 * NQ + 3]
    oq = refs[2 * NQ + 3:]
    acc = None
    for q in range(NQ):
        z = epsr[...] * hq[q][...] + aq[q][...]
        p = jnp.dot(z, W[q * QW:(q + 1) * QW, :],
                    preferred_element_type=jnp.float32)
        acc = p if acc is None else acc + p
    h = _silu(acc + b[...])
    for q in range(NQ):
        oq[q][...] = h[:, q * QW:(q + 1) * QW]


def _conv(hq, aggq, eps_row, W, b):
    return pl.pallas_call(
        _conv_body,
        grid=(N_BLKS,),
        in_specs=(
            [pl.BlockSpec((ROW_BLK, QW), lambda i: (i, 0))] * NQ
            + [pl.BlockSpec((ROW_BLK, QW), lambda i: (i, 0))] * NQ
            + [
                pl.BlockSpec((1, QW), lambda i: (0, 0)),
                pl.BlockSpec((H, H), lambda i: (0, 0)),
                pl.BlockSpec((1, H), lambda i: (0, 0)),
            ]
        ),
        out_specs=[pl.BlockSpec((ROW_BLK, QW), lambda i: (i, 0))] * NQ,
        out_shape=[jax.ShapeDtypeStruct((N, QW), jnp.float32)] * NQ,
    )(*hq, *aggq, eps_row, W, b)


def _pool_body(h0, h1, h2, h3, W1, b1, W2, b2, out, acc):
    i = pl.program_id(0)

    @pl.when(i == 0)
    def _():
        acc[...] = jnp.zeros_like(acc)

    blk = jnp.concatenate([h0[...], h1[...], h2[...], h3[...]], axis=1)
    acc[...] += jnp.sum(blk, axis=0, keepdims=True)

    @pl.when(i == N_BLKS - 1)
    def _():
        g = _silu(acc[...])
        p = _silu(jnp.dot(g, W1[...], preferred_element_type=jnp.float32)
                  + b1[...])
        out[...] = jnp.dot(p, W2[...], preferred_element_type=jnp.float32) \
            + b2[...]


def _pool_mlp(hq, W1, b1, W2p, b2p):
    return pl.pallas_call(
        _pool_body,
        grid=(N_BLKS,),
        in_specs=[pl.BlockSpec((ROW_BLK, QW), lambda i: (i, 0))] * NQ + [
            pl.BlockSpec((H, H // 2), lambda i: (0, 0)),
            pl.BlockSpec((1, H // 2), lambda i: (0, 0)),
            pl.BlockSpec((H // 2, 128), lambda i: (0, 0)),
            pl.BlockSpec((1, 128), lambda i: (0, 0)),
        ],
        out_specs=pl.BlockSpec((1, 128), lambda i: (0, 0)),
        out_shape=jax.ShapeDtypeStruct((1, 128), jnp.float32),
        scratch_shapes=[pltpu.VMEM((1, H), jnp.float32)],
    )(*hq, W1, b1, W2p, b2p)


def kernel(x, edge_index, edge_attr, lin1_W, lin1_b, edgelin_W, edgelin_b,
           eps0, conv0_W, conv0_b, eps1, conv1_W, conv1_b, eps2, conv2_W,
           conv2_b, pred_W1, pred_b1, pred_W2, pred_b2):
    f32 = jnp.float32

    pad = E_PAD - E
    src = jnp.concatenate([edge_index[0], jnp.zeros((pad,), jnp.int32)])
    dst = jnp.concatenate([edge_index[1], jnp.full((pad,), N, jnp.int32)])
    src3 = src.reshape(NTEC, NB, EDGE_BATCH)
    dst3 = dst.reshape(NTEC, NB, EDGE_BATCH)
    ea_pad = jnp.concatenate(
        [edge_attr, jnp.zeros((pad, ED), f32)], axis=0)
    zeros = jnp.zeros((N_OUT, QW), f32)

    lin1_b2 = lin1_b.reshape(1, H)
    edgelin_b2 = edgelin_b.reshape(1, H)
    W2p = jnp.zeros((H // 2, 128), f32).at[:, :O].set(pred_W2)
    b2p = jnp.zeros((1, 128), f32).at[0, :O].set(pred_b2)

    hq = _lin1(x, lin1_W, lin1_b2)
    eq = _edge_lin(ea_pad, edgelin_W, edgelin_b2)

    aggq = _make_sc_round(True)(*hq, src3, dst3, zeros, *eq)
    eps_row = jnp.full((1, QW), 1.0, f32) * (1.0 + eps0)
    hq = _conv(hq, aggq, eps_row, conv0_W, conv0_b.reshape(1, H))

    for epsk, Wk, bk in ((eps1, conv1_W, conv1_b), (eps2, conv2_W, conv2_b)):
        aggq = _make_sc_round(False)(*hq, src3, dst3, zeros, *eq)
        eps_row = jnp.full((1, QW), 1.0, f32) * (1.0 + epsk)
        hq = _conv(hq, aggq, eps_row, Wk, bk.reshape(1, H))

    outp = _pool_mlp(hq, pred_W1, pred_b1.reshape(1, H // 2), W2p, b2p)
    return outp[:, :O]
